# baseline (speedup 1.0000x reference)
import functools

import jax
import jax.numpy as jnp
from jax import lax
from jax.experimental import pallas as pl
from jax.experimental.pallas import tpu as pltpu
from jax.experimental.pallas import tpu_sc as plsc

_NV = 10000
_NE = 10000
_NNZ = 320000
_DIN = 128
_DHID = 64
_NH = 4
_NCLS = 16
_NEG = 0.2

_NC = 2
_NS = 16
_NW = _NC * _NS
_MP = 10112
_ROWS_PER_TILE = _MP // _NS
_PER_W = _NNZ // _NW
_B = 80
_NCHUNK = _PER_W // _B


def _leaky(x):
    return jnp.where(x >= 0, x, _NEG * x)


def _elu(x):
    return jnp.where(x > 0, x, jnp.exp(jnp.minimum(x, 0.0)) - 1.0)


def _sc_phase(table_h, gidx_h, sidx_h, znd_h, av_h, numer_h,
              gbuf, sbuf, rows_v, av_v, sem_i, sem_g, sem_a, sem_s,
              numer_sp, *, D, n_ch, ch_start, weighted, out_base, c, s):
    wid = c * _NS + s
    dw = D - 16 if weighted else D
    bw = dw // n_ch
    nvec = bw // 16

    row0 = s * _ROWS_PER_TILE
    pltpu.sync_copy(znd_h, numer_sp.at[pl.ds(row0, _ROWS_PER_TILE)])
    plsc.subcore_barrier()

    def i_issue(j, b):
        base = wid * _PER_W + j * _B
        pltpu.async_copy(gidx_h.at[pl.ds(base, _B)], gbuf[b], sem_i[b])
        pltpu.async_copy(sidx_h.at[pl.ds(base, _B)], sbuf[b], sem_i[b])

    def g_issue(j, b):
        base = wid * _PER_W + j * _B
        pltpu.make_async_copy(gidx_h.at[pl.ds(base, _B)], gbuf[b],
                              sem_i[b]).wait()
        pltpu.make_async_copy(sidx_h.at[pl.ds(base, _B)], sbuf[b],
                              sem_i[b]).wait()
        pltpu.async_copy(table_h.at[gbuf[b]], rows_v[b], sem_g[b])
        if weighted:
            pltpu.async_copy(av_h.at[sbuf[b]], av_v[b], sem_a[b])

    def g_drain(b):
        pltpu.make_async_copy(table_h.at[gbuf[b]], rows_v[b],
                              sem_g[b]).wait()
        if weighted:
            pltpu.make_async_copy(av_h.at[sbuf[b]], av_v[b],
                                  sem_a[b]).wait()

    def compute(b):
        if not weighted:
            return

        def row(r, rc):
            ae = rows_v[b][r, pl.ds(dw, 16)]
            ex = jnp.exp(_leaky(ae + av_v[b][r]))
            rows_v[b][r, pl.ds(dw, 16)] = ex
            for ch in range(n_ch):
                w = ex[ch_start + ch]
                for j in range(nvec):
                    col = ch * bw + j * 16
                    rows_v[b][r, pl.ds(col, 16)] = (
                        rows_v[b][r, pl.ds(col, 16)] * w)
            return rc

        lax.fori_loop(0, _B, row, 0, unroll=2)

    def s_issue(b):
        pltpu.async_copy(rows_v[b], numer_sp.at[sbuf[b]], sem_s[b],
                         add=True)

    def s_wait(b):
        pltpu.make_async_copy(rows_v[b], numer_sp.at[sbuf[b]],
                              sem_s[b]).wait()

    def step(j, b, do_i=True, do_g=True, do_sw=True):
        bn = (b + 1) % 3
        bp = (b + 2) % 3
        if do_sw:
            s_wait(bp)
        if do_i:
            i_issue(j + 2, bp)
        if do_g:
            g_issue(j + 1, bn)
        g_drain(b)
        compute(b)
        s_issue(b)

    i_issue(0, 0)
    i_issue(1, 1)
    g_issue(0, 0)
    step(0, 0, do_sw=False)

    def triple(i, carry):
        j = 3 * i + 1
        step(j, 1)
        step(j + 1, 2)
        step(j + 2, 0)
        return carry

    lax.fori_loop(0, (_NCHUNK - 5) // 3, triple, 0)
    step(_NCHUNK - 4, 1)
    step(_NCHUNK - 3, 2)
    step(_NCHUNK - 2, 0, do_i=False)
    step(_NCHUNK - 1, 1, do_i=False, do_g=False)
    s_wait(1)

    plsc.subcore_barrier()
    out0 = out_base + c * _MP + row0
    pltpu.sync_copy(numer_sp.at[pl.ds(row0, _ROWS_PER_TILE)],
                    numer_h.at[pl.ds(out0, _ROWS_PER_TILE)])


@functools.lru_cache(maxsize=None)
def _make_sc_multi(D, specs):
    n_tab = max(sp[0] for sp in specs) + 1
    av_idxs = [sp[1] for sp in specs if sp[1] is not None]
    n_av = (max(av_idxs) + 1) if av_idxs else 0
    nph = len(specs)
    mesh = plsc.VectorSubcoreMesh(core_axis_name="c", subcore_axis_name="s")

    def body(*refs):
        tabs = refs[:n_tab]
        gidx_h, sidx_h, znd_h = refs[n_tab:n_tab + 3]
        avs = refs[n_tab + 3:n_tab + 3 + n_av]
        numer_h = refs[n_tab + 3 + n_av]
        scr = refs[n_tab + 4 + n_av:]
        c = lax.axis_index("c")
        s = lax.axis_index("s")
        for p, (ti, ai, n_ch, ch_start) in enumerate(specs):
            _sc_phase(tabs[ti], gidx_h, sidx_h, znd_h,
                      avs[ai] if ai is not None else None, numer_h, *scr,
                      D=D, n_ch=n_ch, ch_start=ch_start,
                      weighted=ai is not None, out_base=p * _NC * _MP,
                      c=c, s=s)

    f = pl.kernel(
        body,
        out_type=jax.ShapeDtypeStruct((nph * _NC * _MP, D), jnp.float32),
        mesh=mesh,
        scratch_types=[
            [pltpu.VMEM((_B,), jnp.int32) for _ in range(3)],
            [pltpu.VMEM((_B,), jnp.int32) for _ in range(3)],
            [pltpu.VMEM((_B, D), jnp.float32) for _ in range(3)],
            [pltpu.VMEM((_B, 16), jnp.float32) for _ in range(3)],
            [pltpu.SemaphoreType.DMA for _ in range(3)],
            [pltpu.SemaphoreType.DMA for _ in range(3)],
            [pltpu.SemaphoreType.DMA for _ in range(3)],
            [pltpu.SemaphoreType.DMA for _ in range(3)],
            pltpu.VMEM_SHARED((_MP, D), jnp.float32),
        ],
        compiler_params=pltpu.CompilerParams(use_tc_tiling_on_sc=False),
    )

    def run(tables, gidx, sidx, avs):
        znd = jnp.zeros((_ROWS_PER_TILE, D), jnp.float32)
        numer = f(*tables, gidx, sidx, znd, *avs)
        return numer.reshape(nph * _NC, _MP, D)

    return run


_BR = 2000


def _tc_prep(Wcat, bcat, BDd, BDe):
    def body(w_r, b_r, dd_r, de_r, pv_r, qv_r, pe_r, qe_r):
        w = w_r[...]
        b = b_r[...]
        pv_r[...] = jnp.dot(w, dd_r[...], preferred_element_type=jnp.float32)
        qv_r[...] = jnp.dot(b, dd_r[...], preferred_element_type=jnp.float32)
        pe_r[...] = jnp.dot(w, de_r[...], preferred_element_type=jnp.float32)
        qe_r[...] = jnp.dot(b, de_r[...], preferred_element_type=jnp.float32)

    return pl.pallas_call(
        body,
        out_shape=[jax.ShapeDtypeStruct((_DIN, 16), jnp.float32),
                   jax.ShapeDtypeStruct((1, 16), jnp.float32),
                   jax.ShapeDtypeStruct((_DIN, 16), jnp.float32),
                   jax.ShapeDtypeStruct((1, 16), jnp.float32)],
    )(Wcat, bcat, BDd, BDe)


def _tc_matvec(x, P, q):
    def body(x_r, p_r, q_r, o_r):
        o_r[...] = (jnp.dot(x_r[0], p_r[...],
                            preferred_element_type=jnp.float32)
                    + q_r[...])[None]

    grid = (x.shape[0], _NV // _BR)
    return pl.pallas_call(
        body,
        grid=grid,
        in_specs=[pl.BlockSpec((1, _BR, _DIN), lambda k, i: (k, i, 0)),
                  pl.BlockSpec((_DIN, 16), lambda k, i: (0, 0)),
                  pl.BlockSpec((1, 16), lambda k, i: (0, 0))],
        out_specs=pl.BlockSpec((1, _BR, 16), lambda k, i: (k, i, 0)),
        out_shape=jax.ShapeDtypeStruct((x.shape[0], _NV, 16), jnp.float32),
    )(x, P, q)


def _tc_edge(aggP, Wcat, bcat, PE, qE):
    def body(a_r, w_r, b_r, pe_r, qe_r, y0_r, y1_r):
        full = a_r[0] + a_r[1]
        cnt = jnp.maximum(full[:, 128:129], 1.0)
        yagg = full[:, :128] / cnt
        y = jnp.dot(yagg, w_r[...], preferred_element_type=jnp.float32) + b_r[...]
        ae = jnp.dot(yagg, pe_r[...],
                     preferred_element_type=jnp.float32) + qe_r[...]
        y0_r[...] = jnp.concatenate([y[:, :128], ae], axis=-1)
        y1_r[...] = jnp.concatenate([y[:, 128:], ae], axis=-1)

    grid = (_NE // _BR,)
    return pl.pallas_call(
        body,
        grid=grid,
        in_specs=[pl.BlockSpec((2, _BR, 144), lambda i: (0, i, 0)),
                  pl.BlockSpec((128, 256), lambda i: (0, 0)),
                  pl.BlockSpec((1, 256), lambda i: (0, 0)),
                  pl.BlockSpec((128, 16), lambda i: (0, 0)),
                  pl.BlockSpec((1, 16), lambda i: (0, 0))],
        out_specs=[pl.BlockSpec((_BR, 144), lambda i: (i, 0)),
                   pl.BlockSpec((_BR, 144), lambda i: (i, 0))],
        out_shape=[jax.ShapeDtypeStruct((_NE, 144), jnp.float32),
                   jax.ShapeDtypeStruct((_NE, 144), jnp.float32)],
    )(aggP, Wcat, bcat, PE, qE)


def _tc_l2vert(numerP, W2, b2, oad):
    def body(n_r, w_r, b_r, ad_r, x2_r, av_r):
        blocks = []
        for p in range(4):
            full = n_r[2 * p] + n_r[2 * p + 1]
            half = p % 2
            for cch in range(2):
                lane = 128 + 2 * half + cch
                dcol = jnp.maximum(full[:, lane:lane + 1], 1e-12)
                blocks.append(_elu(full[:, 64 * cch: 64 * cch + 64] / dcol))
        out1 = jnp.concatenate(blocks, axis=-1)
        x2 = jnp.dot(out1, w_r[...], preferred_element_type=jnp.float32) + b_r[...]
        x2_r[...] = x2
        av = jnp.sum(x2 * ad_r[...], axis=-1, keepdims=True)
        av_r[...] = jnp.concatenate(
            [av, jnp.zeros((av.shape[0], 15), jnp.float32)], axis=-1)

    grid = (_NV // _BR,)
    return pl.pallas_call(
        body,
        grid=grid,
        in_specs=[pl.BlockSpec((8, _BR, 144), lambda i: (0, i, 0)),
                  pl.BlockSpec((512, 16), lambda i: (0, 0)),
                  pl.BlockSpec((1, 16), lambda i: (0, 0)),
                  pl.BlockSpec((1, 16), lambda i: (0, 0))],
        out_specs=[pl.BlockSpec((_BR, 16), lambda i: (i, 0)),
                   pl.BlockSpec((_BR, 16), lambda i: (i, 0))],
        out_shape=[jax.ShapeDtypeStruct((_NV, 16), jnp.float32),
                   jax.ShapeDtypeStruct((_NV, 16), jnp.float32)],
    )(numerP, W2, b2, oad)


def _tc_l2edge(agg2P, cntP, oae):
    def body(a_r, c_r, ae_w, yt_r):
        cnt = jnp.maximum(c_r[0, :, 0:1] + c_r[1, :, 0:1], 1.0)
        y2 = (a_r[0] + a_r[1]) / cnt
        ae = jnp.sum(y2 * ae_w[...], axis=-1, keepdims=True)
        yt_r[...] = jnp.concatenate(
            [y2, ae, jnp.zeros((ae.shape[0], 15), jnp.float32)], axis=-1)

    grid = (_NE // _BR,)
    return pl.pallas_call(
        body,
        grid=grid,
        in_specs=[pl.BlockSpec((2, _BR, 16), lambda i: (0, i, 0)),
                  pl.BlockSpec((2, _BR, 16), lambda i: (0, i, 0)),
                  pl.BlockSpec((1, 16), lambda i: (0, 0))],
        out_specs=pl.BlockSpec((_BR, 32), lambda i: (i, 0)),
        out_shape=jax.ShapeDtypeStruct((_NE, 32), jnp.float32),
    )(agg2P, cntP, oae)


def _tc_final(numer2P):
    def body(n_r, o_r):
        full = n_r[0] + n_r[1]
        den = jnp.maximum(full[:, 16:17], 1e-12)
        o_r[...] = _elu(full[:, :16] / den)

    grid = (_NV // _BR,)
    return pl.pallas_call(
        body,
        grid=grid,
        in_specs=[pl.BlockSpec((2, _BR, 32), lambda i: (0, i, 0))],
        out_specs=pl.BlockSpec((_BR, 16), lambda i: (i, 0)),
        out_shape=jax.ShapeDtypeStruct((_NV, _NCLS), jnp.float32),
    )(numer2P)


def kernel(x_list, hg, heads_theta_w, heads_theta_b, heads_att_e,
           heads_att_dst, out_theta_w, out_theta_b, out_att_e, out_att_dst):
    v_idx, e_idx = hg[0], hg[1]

    Wcat = jnp.concatenate([heads_theta_w[h] for h in range(_NH)], axis=1)
    bcat = heads_theta_b.reshape(1, _NH * _DHID)
    blkmask = jnp.kron(jnp.eye(_NH, dtype=jnp.float32),
                       jnp.ones((_DHID, 1), jnp.float32))
    BDd = jnp.pad(heads_att_dst.reshape(-1, 1) * blkmask, ((0, 0), (0, 12)))
    BDe = jnp.pad(heads_att_e.reshape(-1, 1) * blkmask, ((0, 0), (0, 12)))
    PV, qV, PE, qE = _tc_prep(Wcat, bcat, BDd, BDe)

    alphaV = _tc_matvec(x_list, PV, qV)

    ones_pad = jnp.concatenate(
        [jnp.ones((2, _NV, 1), jnp.float32),
         jnp.zeros((2, _NV, 15), jnp.float32)], axis=-1)
    x_aug = jnp.concatenate([x_list, ones_pad], axis=-1)

    v2e = _make_sc_multi(144, ((0, None, 1, 0), (1, None, 1, 0)))
    aggPall = v2e((x_aug[0], x_aug[1]), v_idx, e_idx, ())
    ytabs = []
    for k in range(2):
        y0, y1 = _tc_edge(aggPall[2 * k:2 * k + 2], Wcat, bcat, PE, qE)
        ytabs += [y0, y1]
    e2v = _make_sc_multi(144, ((0, 0, 2, 0), (1, 0, 2, 2),
                               (2, 1, 2, 0), (3, 1, 2, 2)))
    numerP = e2v(tuple(ytabs), e_idx, v_idx,
                 (alphaV[0], alphaV[1]))[:, :_NV]

    X2, aV2 = _tc_l2vert(numerP, out_theta_w, out_theta_b.reshape(1, -1),
                         out_att_dst.reshape(1, -1))
    v2e2 = _make_sc_multi(16, ((0, None, 1, 0),))
    agg2P = v2e2((X2,), v_idx, e_idx, ())
    ytab2 = _tc_l2edge(agg2P[:, :_NE], aggPall[0:2, :_NE, 128:144],
                       out_att_e.reshape(1, -1))
    e2v2 = _make_sc_multi(32, ((0, 0, 1, 0),))
    n2P = e2v2((ytab2,), e_idx, v_idx, (aV2,))
    return _tc_final(n2P[:, :_NV])


# speedup vs baseline: 1.0866x; 1.0866x over previous
import functools

import jax
import jax.numpy as jnp
from jax import lax
from jax.experimental import pallas as pl
from jax.experimental.pallas import tpu as pltpu
from jax.experimental.pallas import tpu_sc as plsc

_NV = 10000
_NE = 10000
_NNZ = 320000
_DIN = 128
_DHID = 64
_NH = 4
_NCLS = 16
_NEG = 0.2

_NC = 2
_NS = 16
_NW = _NC * _NS
_MP = 10112
_ROWS_PER_TILE = _MP // _NS
_PER_W = _NNZ // _NW
_B = 80
_NCHUNK = _PER_W // _B


def _leaky(x):
    return jnp.where(x >= 0, x, _NEG * x)


def _elu(x):
    return jnp.where(x > 0, x, jnp.exp(jnp.minimum(x, 0.0)) - 1.0)


def _sc_phase(table_h, gidx_h, sidx_h, znd_h, av_h, numer_h,
              gbuf, sbuf, rows_v, av_v, sem_i, sem_g, sem_a, sem_s,
              numer_sp, *, D, n_ch, ch_start, weighted, out_base, c, s):
    wid = c * _NS + s
    dw = D - 16 if weighted else D
    bw = dw // n_ch
    nvec = bw // 16

    row0 = s * _ROWS_PER_TILE
    pltpu.sync_copy(znd_h, numer_sp.at[pl.ds(row0, _ROWS_PER_TILE)])
    plsc.subcore_barrier()

    def i_issue(j, b):
        base = wid * _PER_W + j * _B
        pltpu.async_copy(gidx_h.at[pl.ds(base, _B)], gbuf[b], sem_i[b])
        pltpu.async_copy(sidx_h.at[pl.ds(base, _B)], sbuf[b], sem_i[b])

    def g_issue(j, b):
        base = wid * _PER_W + j * _B
        pltpu.make_async_copy(gidx_h.at[pl.ds(base, _B)], gbuf[b],
                              sem_i[b]).wait()
        pltpu.make_async_copy(sidx_h.at[pl.ds(base, _B)], sbuf[b],
                              sem_i[b]).wait()
        pltpu.async_copy(table_h.at[gbuf[b]], rows_v[b], sem_g[b])
        if weighted:
            pltpu.async_copy(av_h.at[sbuf[b]], av_v[b], sem_a[b])

    def g_drain(b):
        pltpu.make_async_copy(table_h.at[gbuf[b]], rows_v[b],
                              sem_g[b]).wait()
        if weighted:
            pltpu.make_async_copy(av_h.at[sbuf[b]], av_v[b],
                                  sem_a[b]).wait()

    def compute(b):
        if not weighted:
            return

        def row(r, rc):
            ae = rows_v[b][r, pl.ds(dw, 16)]
            ex = jnp.exp(_leaky(ae + av_v[b][r]))
            rows_v[b][r, pl.ds(dw, 16)] = ex
            for ch in range(n_ch):
                w = ex[ch_start + ch]
                for j in range(nvec):
                    col = ch * bw + j * 16
                    rows_v[b][r, pl.ds(col, 16)] = (
                        rows_v[b][r, pl.ds(col, 16)] * w)
            return rc

        lax.fori_loop(0, _B, row, 0, unroll=2)

    def s_issue(b):
        pltpu.async_copy(rows_v[b], numer_sp.at[sbuf[b]], sem_s[b],
                         add=True)

    def s_wait(b):
        pltpu.make_async_copy(rows_v[b], numer_sp.at[sbuf[b]],
                              sem_s[b]).wait()

    def step(j, b, do_i=True, do_g=True, do_sw=True):
        bn = (b + 1) % 3
        bp = (b + 2) % 3
        if do_sw:
            s_wait(bp)
        if do_i:
            i_issue(j + 2, bp)
        if do_g:
            g_issue(j + 1, bn)
        g_drain(b)
        compute(b)
        s_issue(b)

    i_issue(0, 0)
    i_issue(1, 1)
    g_issue(0, 0)
    step(0, 0, do_sw=False)

    def triple(i, carry):
        j = 3 * i + 1
        step(j, 1)
        step(j + 1, 2)
        step(j + 2, 0)
        return carry

    lax.fori_loop(0, (_NCHUNK - 5) // 3, triple, 0)
    step(_NCHUNK - 4, 1)
    step(_NCHUNK - 3, 2)
    step(_NCHUNK - 2, 0, do_i=False)
    step(_NCHUNK - 1, 1, do_i=False, do_g=False)
    s_wait(1)

    plsc.subcore_barrier()
    out0 = out_base + c * _MP + row0
    pltpu.sync_copy(numer_sp.at[pl.ds(row0, _ROWS_PER_TILE)],
                    numer_h.at[pl.ds(out0, _ROWS_PER_TILE)])


@functools.lru_cache(maxsize=None)
def _make_sc_multi(D, specs):
    n_tab = max(sp[0] for sp in specs) + 1
    av_idxs = [sp[1] for sp in specs if sp[1] is not None]
    n_av = (max(av_idxs) + 1) if av_idxs else 0
    nph = len(specs)
    mesh = plsc.VectorSubcoreMesh(core_axis_name="c", subcore_axis_name="s")

    def body(*refs):
        tabs = refs[:n_tab]
        gidx_h, sidx_h, znd_h = refs[n_tab:n_tab + 3]
        avs = refs[n_tab + 3:n_tab + 3 + n_av]
        numer_h = refs[n_tab + 3 + n_av]
        scr = refs[n_tab + 4 + n_av:]
        c = lax.axis_index("c")
        s = lax.axis_index("s")
        for p, (ti, ai, n_ch, ch_start) in enumerate(specs):
            _sc_phase(tabs[ti], gidx_h, sidx_h, znd_h,
                      avs[ai] if ai is not None else None, numer_h, *scr,
                      D=D, n_ch=n_ch, ch_start=ch_start,
                      weighted=ai is not None, out_base=p * _NC * _MP,
                      c=c, s=s)

    f = pl.kernel(
        body,
        out_type=jax.ShapeDtypeStruct((nph * _NC * _MP, D), jnp.float32),
        mesh=mesh,
        scratch_types=[
            [pltpu.VMEM((_B,), jnp.int32) for _ in range(3)],
            [pltpu.VMEM((_B,), jnp.int32) for _ in range(3)],
            [pltpu.VMEM((_B, D), jnp.float32) for _ in range(3)],
            [pltpu.VMEM((_B, 16), jnp.float32) for _ in range(3)],
            [pltpu.SemaphoreType.DMA for _ in range(3)],
            [pltpu.SemaphoreType.DMA for _ in range(3)],
            [pltpu.SemaphoreType.DMA for _ in range(3)],
            [pltpu.SemaphoreType.DMA for _ in range(3)],
            pltpu.VMEM_SHARED((_MP, D), jnp.float32),
        ],
        compiler_params=pltpu.CompilerParams(use_tc_tiling_on_sc=False),
    )

    def run(tables, gidx, sidx, avs):
        znd = jnp.zeros((_ROWS_PER_TILE, D), jnp.float32)
        numer = f(*tables, gidx, sidx, znd, *avs)
        return numer.reshape(nph * _NC, _MP, D)

    return run


_BR = 2000


def _tc_prep(Wcat, bcat, BDd, BDe):
    def body(w_r, b_r, dd_r, de_r, pv_r, qv_r, pe_r, qe_r):
        w = w_r[...]
        b = b_r[...]
        pv_r[...] = jnp.dot(w, dd_r[...], preferred_element_type=jnp.float32)
        qv_r[...] = jnp.dot(b, dd_r[...], preferred_element_type=jnp.float32)
        pe_r[...] = jnp.dot(w, de_r[...], preferred_element_type=jnp.float32)
        qe_r[...] = jnp.dot(b, de_r[...], preferred_element_type=jnp.float32)

    return pl.pallas_call(
        body,
        out_shape=[jax.ShapeDtypeStruct((_DIN, 16), jnp.float32),
                   jax.ShapeDtypeStruct((1, 16), jnp.float32),
                   jax.ShapeDtypeStruct((_DIN, 16), jnp.float32),
                   jax.ShapeDtypeStruct((1, 16), jnp.float32)],
    )(Wcat, bcat, BDd, BDe)


def _tc_matvec(x, P, q):
    def body(x_r, p_r, q_r, o_r):
        o_r[...] = (jnp.dot(x_r[0], p_r[...],
                            preferred_element_type=jnp.float32)
                    + q_r[...])[None]

    grid = (x.shape[0], _NV // _BR)
    return pl.pallas_call(
        body,
        grid=grid,
        in_specs=[pl.BlockSpec((1, _BR, _DIN), lambda k, i: (k, i, 0)),
                  pl.BlockSpec((_DIN, 16), lambda k, i: (0, 0)),
                  pl.BlockSpec((1, 16), lambda k, i: (0, 0))],
        out_specs=pl.BlockSpec((1, _BR, 16), lambda k, i: (k, i, 0)),
        out_shape=jax.ShapeDtypeStruct((x.shape[0], _NV, 16), jnp.float32),
    )(x, P, q)


def _tc_edge(aggP, Wcat, bcat, PE, qE):
    def body(a_r, w_r, b_r, pe_r, qe_r, y0_r, y1_r):
        full = a_r[0] + a_r[1]
        cnt = jnp.maximum(full[:, 128:129], 1.0)
        yagg = full[:, :128] / cnt
        y = jnp.dot(yagg, w_r[...], preferred_element_type=jnp.float32) + b_r[...]
        ae = jnp.dot(yagg, pe_r[...],
                     preferred_element_type=jnp.float32) + qe_r[...]
        y0_r[...] = jnp.concatenate([y[:, :128], ae], axis=-1)
        y1_r[...] = jnp.concatenate([y[:, 128:], ae], axis=-1)

    grid = (_NE // _BR,)
    return pl.pallas_call(
        body,
        grid=grid,
        in_specs=[pl.BlockSpec((2, _BR, 144), lambda i: (0, i, 0)),
                  pl.BlockSpec((128, 256), lambda i: (0, 0)),
                  pl.BlockSpec((1, 256), lambda i: (0, 0)),
                  pl.BlockSpec((128, 16), lambda i: (0, 0)),
                  pl.BlockSpec((1, 16), lambda i: (0, 0))],
        out_specs=[pl.BlockSpec((_BR, 144), lambda i: (i, 0)),
                   pl.BlockSpec((_BR, 144), lambda i: (i, 0))],
        out_shape=[jax.ShapeDtypeStruct((_NE, 144), jnp.float32),
                   jax.ShapeDtypeStruct((_NE, 144), jnp.float32)],
    )(aggP, Wcat, bcat, PE, qE)


def _tc_l2vert(numerP, W2, b2, oad):
    def body(n_r, w_r, b_r, ad_r, x2_r, av_r):
        blocks = []
        for p in range(4):
            full = n_r[2 * p] + n_r[2 * p + 1]
            half = p % 2
            for cch in range(2):
                lane = 128 + 2 * half + cch
                dcol = jnp.maximum(full[:, lane:lane + 1], 1e-12)
                blocks.append(_elu(full[:, 64 * cch: 64 * cch + 64] / dcol))
        out1 = jnp.concatenate(blocks, axis=-1)
        x2 = jnp.dot(out1, w_r[...], preferred_element_type=jnp.float32) + b_r[...]
        x2_r[...] = x2
        av = jnp.sum(x2 * ad_r[...], axis=-1, keepdims=True)
        av_r[...] = jnp.concatenate(
            [av, jnp.zeros((av.shape[0], 15), jnp.float32)], axis=-1)

    grid = (_NV // _BR,)
    return pl.pallas_call(
        body,
        grid=grid,
        in_specs=[pl.BlockSpec((8, _BR, 144), lambda i: (0, i, 0)),
                  pl.BlockSpec((512, 16), lambda i: (0, 0)),
                  pl.BlockSpec((1, 16), lambda i: (0, 0)),
                  pl.BlockSpec((1, 16), lambda i: (0, 0))],
        out_specs=[pl.BlockSpec((_BR, 16), lambda i: (i, 0)),
                   pl.BlockSpec((_BR, 16), lambda i: (i, 0))],
        out_shape=[jax.ShapeDtypeStruct((_NV, 16), jnp.float32),
                   jax.ShapeDtypeStruct((_NV, 16), jnp.float32)],
    )(numerP, W2, b2, oad)


def _tc_l2edge(agg2P, cntP, oae):
    def body(a_r, c_r, ae_w, yt_r):
        cnt = jnp.maximum(c_r[0, :, 0:1] + c_r[1, :, 0:1], 1.0)
        y2 = (a_r[0] + a_r[1]) / cnt
        ae = jnp.sum(y2 * ae_w[...], axis=-1, keepdims=True)
        yt_r[...] = jnp.concatenate(
            [y2, ae, jnp.zeros((ae.shape[0], 15), jnp.float32)], axis=-1)

    grid = (_NE // _BR,)
    return pl.pallas_call(
        body,
        grid=grid,
        in_specs=[pl.BlockSpec((2, _BR, 16), lambda i: (0, i, 0)),
                  pl.BlockSpec((2, _BR, 16), lambda i: (0, i, 0)),
                  pl.BlockSpec((1, 16), lambda i: (0, 0))],
        out_specs=pl.BlockSpec((_BR, 32), lambda i: (i, 0)),
        out_shape=jax.ShapeDtypeStruct((_NE, 32), jnp.float32),
    )(agg2P, cntP, oae)


def _tc_final(numer2P):
    def body(n_r, o_r):
        full = n_r[0] + n_r[1]
        den = jnp.maximum(full[:, 16:17], 1e-12)
        o_r[...] = _elu(full[:, :16] / den)

    grid = (_NV // _BR,)
    return pl.pallas_call(
        body,
        grid=grid,
        in_specs=[pl.BlockSpec((2, _BR, 32), lambda i: (0, i, 0))],
        out_specs=pl.BlockSpec((_BR, 16), lambda i: (i, 0)),
        out_shape=jax.ShapeDtypeStruct((_NV, _NCLS), jnp.float32),
    )(numer2P)


def kernel(x_list, hg, heads_theta_w, heads_theta_b, heads_att_e,
           heads_att_dst, out_theta_w, out_theta_b, out_att_e, out_att_dst):
    v_idx, e_idx = hg[0], hg[1]

    Wcat = jnp.concatenate([heads_theta_w[h] for h in range(_NH)], axis=1)
    bcat = heads_theta_b.reshape(1, _NH * _DHID)
    blkmask = jnp.kron(jnp.eye(_NH, dtype=jnp.float32),
                       jnp.ones((_DHID, 1), jnp.float32))
    BDd = jnp.pad(heads_att_dst.reshape(-1, 1) * blkmask, ((0, 0), (0, 12)))
    BDe = jnp.pad(heads_att_e.reshape(-1, 1) * blkmask, ((0, 0), (0, 12)))
    PV, qV, PE, qE = _tc_prep(Wcat, bcat, BDd, BDe)

    alphaV = _tc_matvec(x_list, PV, qV)

    ones_pad = jnp.concatenate(
        [jnp.ones((2, _NV, 1), jnp.float32),
         jnp.zeros((2, _NV, 15), jnp.float32)], axis=-1)
    x_aug = jnp.concatenate([x_list, ones_pad], axis=-1)

    v2e = _make_sc_multi(144, ((0, None, 1, 0),))
    numer_parts = []
    aggP0 = None
    for k in range(2):
        aggP = v2e((x_aug[k],), v_idx, e_idx, ())
        if aggP0 is None:
            aggP0 = aggP
        y0, y1 = _tc_edge(aggP, Wcat, bcat, PE, qE)
        for half, ytab in enumerate((y0, y1)):
            e2v = _make_sc_multi(144, ((0, 0, 2, 2 * half),))
            numer_parts.append(e2v((ytab,), e_idx, v_idx,
                                   (alphaV[k],))[:, :_NV])

    numerP = jnp.concatenate(numer_parts, axis=0)

    X2, aV2 = _tc_l2vert(numerP, out_theta_w, out_theta_b.reshape(1, -1),
                         out_att_dst.reshape(1, -1))
    v2e2 = _make_sc_multi(16, ((0, None, 1, 0),))
    agg2P = v2e2((X2,), v_idx, e_idx, ())
    ytab2 = _tc_l2edge(agg2P[:, :_NE], aggP0[:, :_NE, 128:144],
                       out_att_e.reshape(1, -1))
    e2v2 = _make_sc_multi(32, ((0, 0, 1, 0),))
    n2P = e2v2((ytab2,), e_idx, v_idx, (aV2,))
    return _tc_final(n2P[:, :_NV])


# speedup vs baseline: 1.0968x; 1.0094x over previous
import functools

import jax
import jax.numpy as jnp
from jax import lax
from jax.experimental import pallas as pl
from jax.experimental.pallas import tpu as pltpu
from jax.experimental.pallas import tpu_sc as plsc

_NV = 10000
_NE = 10000
_NNZ = 320000
_DIN = 128
_DHID = 64
_NH = 4
_NCLS = 16
_NEG = 0.2

_NC = 2
_NS = 16
_NW = _NC * _NS
_MP = 10112
_ROWS_PER_TILE = _MP // _NS
_PER_W = _NNZ // _NW
_B = 80
_NCHUNK = _PER_W // _B


def _leaky(x):
    return jnp.where(x >= 0, x, _NEG * x)


def _elu(x):
    return jnp.where(x > 0, x, jnp.exp(jnp.minimum(x, 0.0)) - 1.0)


def _sc_phase(table_h, gidx_h, sidx_h, znd_h, av_h, numer_h,
              gbuf, sbuf, rows_v, av_v, sem_i, sem_g, sem_a, sem_s,
              numer_sp, *, D, n_ch, ch_start, weighted, out_base, c, s):
    wid = c * _NS + s
    dw = D - 16 if weighted else D
    bw = dw // n_ch
    nvec = bw // 16

    row0 = s * _ROWS_PER_TILE
    pltpu.sync_copy(znd_h, numer_sp.at[pl.ds(row0, _ROWS_PER_TILE)])
    plsc.subcore_barrier()

    def i_issue(j, b):
        base = wid * _PER_W + j * _B
        pltpu.async_copy(gidx_h.at[pl.ds(base, _B)], gbuf[b], sem_i[b])
        pltpu.async_copy(sidx_h.at[pl.ds(base, _B)], sbuf[b], sem_i[b])

    def g_issue(j, b):
        base = wid * _PER_W + j * _B
        pltpu.make_async_copy(gidx_h.at[pl.ds(base, _B)], gbuf[b],
                              sem_i[b]).wait()
        pltpu.make_async_copy(sidx_h.at[pl.ds(base, _B)], sbuf[b],
                              sem_i[b]).wait()
        pltpu.async_copy(table_h.at[gbuf[b]], rows_v[b], sem_g[b])
        if weighted:
            pltpu.async_copy(av_h.at[sbuf[b]], av_v[b], sem_a[b])

    def g_drain(b):
        pltpu.make_async_copy(table_h.at[gbuf[b]], rows_v[b],
                              sem_g[b]).wait()
        if weighted:
            pltpu.make_async_copy(av_h.at[sbuf[b]], av_v[b],
                                  sem_a[b]).wait()

    def compute(b):
        if not weighted:
            return

        def row(r, rc):
            ae = rows_v[b][r, pl.ds(dw, 16)]
            ex = jnp.exp(_leaky(ae + av_v[b][r]))
            rows_v[b][r, pl.ds(dw, 16)] = ex
            for ch in range(n_ch):
                w = ex[ch_start + ch]
                for j in range(nvec):
                    col = ch * bw + j * 16
                    rows_v[b][r, pl.ds(col, 16)] = (
                        rows_v[b][r, pl.ds(col, 16)] * w)
            return rc

        lax.fori_loop(0, _B, row, 0, unroll=4)

    def s_issue(b):
        pltpu.async_copy(rows_v[b], numer_sp.at[sbuf[b]], sem_s[b],
                         add=True)

    def s_wait(b):
        pltpu.make_async_copy(rows_v[b], numer_sp.at[sbuf[b]],
                              sem_s[b]).wait()

    def step(j, b, do_i=True, do_g=True, do_sw=True):
        bn = (b + 1) % 3
        bp = (b + 2) % 3
        if do_g:
            g_issue(j + 1, bn)
        if do_sw:
            s_wait(bp)
        if do_i:
            i_issue(j + 2, bp)
        g_drain(b)
        compute(b)
        s_issue(b)

    i_issue(0, 0)
    i_issue(1, 1)
    g_issue(0, 0)
    step(0, 0, do_sw=False)

    def triple(i, carry):
        j = 3 * i + 1
        step(j, 1)
        step(j + 1, 2)
        step(j + 2, 0)
        return carry

    lax.fori_loop(0, (_NCHUNK - 5) // 3, triple, 0)
    step(_NCHUNK - 4, 1)
    step(_NCHUNK - 3, 2)
    step(_NCHUNK - 2, 0, do_i=False)
    step(_NCHUNK - 1, 1, do_i=False, do_g=False)
    s_wait(1)

    plsc.subcore_barrier()
    out0 = out_base + c * _MP + row0
    pltpu.sync_copy(numer_sp.at[pl.ds(row0, _ROWS_PER_TILE)],
                    numer_h.at[pl.ds(out0, _ROWS_PER_TILE)])


@functools.lru_cache(maxsize=None)
def _make_sc_multi(D, specs):
    n_tab = max(sp[0] for sp in specs) + 1
    av_idxs = [sp[1] for sp in specs if sp[1] is not None]
    n_av = (max(av_idxs) + 1) if av_idxs else 0
    nph = len(specs)
    mesh = plsc.VectorSubcoreMesh(core_axis_name="c", subcore_axis_name="s")

    def body(*refs):
        tabs = refs[:n_tab]
        gidx_h, sidx_h, znd_h = refs[n_tab:n_tab + 3]
        avs = refs[n_tab + 3:n_tab + 3 + n_av]
        numer_h = refs[n_tab + 3 + n_av]
        scr = refs[n_tab + 4 + n_av:]
        c = lax.axis_index("c")
        s = lax.axis_index("s")
        for p, (ti, ai, n_ch, ch_start) in enumerate(specs):
            _sc_phase(tabs[ti], gidx_h, sidx_h, znd_h,
                      avs[ai] if ai is not None else None, numer_h, *scr,
                      D=D, n_ch=n_ch, ch_start=ch_start,
                      weighted=ai is not None, out_base=p * _NC * _MP,
                      c=c, s=s)

    f = pl.kernel(
        body,
        out_type=jax.ShapeDtypeStruct((nph * _NC * _MP, D), jnp.float32),
        mesh=mesh,
        scratch_types=[
            [pltpu.VMEM((_B,), jnp.int32) for _ in range(3)],
            [pltpu.VMEM((_B,), jnp.int32) for _ in range(3)],
            [pltpu.VMEM((_B, D), jnp.float32) for _ in range(3)],
            [pltpu.VMEM((_B, 16), jnp.float32) for _ in range(3)],
            [pltpu.SemaphoreType.DMA for _ in range(3)],
            [pltpu.SemaphoreType.DMA for _ in range(3)],
            [pltpu.SemaphoreType.DMA for _ in range(3)],
            [pltpu.SemaphoreType.DMA for _ in range(3)],
            pltpu.VMEM_SHARED((_MP, D), jnp.float32),
        ],
        compiler_params=pltpu.CompilerParams(use_tc_tiling_on_sc=False),
    )

    def run(tables, gidx, sidx, avs):
        znd = jnp.zeros((_ROWS_PER_TILE, D), jnp.float32)
        numer = f(*tables, gidx, sidx, znd, *avs)
        return numer.reshape(nph * _NC, _MP, D)

    return run


_BR = 2000


def _tc_prep(Wcat, bcat, BDd, BDe):
    def body(w_r, b_r, dd_r, de_r, pv_r, qv_r, pe_r, qe_r):
        w = w_r[...]
        b = b_r[...]
        pv_r[...] = jnp.dot(w, dd_r[...], preferred_element_type=jnp.float32)
        qv_r[...] = jnp.dot(b, dd_r[...], preferred_element_type=jnp.float32)
        pe_r[...] = jnp.dot(w, de_r[...], preferred_element_type=jnp.float32)
        qe_r[...] = jnp.dot(b, de_r[...], preferred_element_type=jnp.float32)

    return pl.pallas_call(
        body,
        out_shape=[jax.ShapeDtypeStruct((_DIN, 16), jnp.float32),
                   jax.ShapeDtypeStruct((1, 16), jnp.float32),
                   jax.ShapeDtypeStruct((_DIN, 16), jnp.float32),
                   jax.ShapeDtypeStruct((1, 16), jnp.float32)],
    )(Wcat, bcat, BDd, BDe)


def _tc_matvec(x, P, q):
    def body(x_r, p_r, q_r, o_r):
        o_r[...] = (jnp.dot(x_r[0], p_r[...],
                            preferred_element_type=jnp.float32)
                    + q_r[...])[None]

    grid = (x.shape[0], _NV // _BR)
    return pl.pallas_call(
        body,
        grid=grid,
        in_specs=[pl.BlockSpec((1, _BR, _DIN), lambda k, i: (k, i, 0)),
                  pl.BlockSpec((_DIN, 16), lambda k, i: (0, 0)),
                  pl.BlockSpec((1, 16), lambda k, i: (0, 0))],
        out_specs=pl.BlockSpec((1, _BR, 16), lambda k, i: (k, i, 0)),
        out_shape=jax.ShapeDtypeStruct((x.shape[0], _NV, 16), jnp.float32),
    )(x, P, q)


def _tc_edge(aggP, Wcat, bcat, PE, qE):
    def body(a_r, w_r, b_r, pe_r, qe_r, y0_r, y1_r):
        full = a_r[0] + a_r[1]
        cnt = jnp.maximum(full[:, 128:129], 1.0)
        yagg = full[:, :128] / cnt
        y = jnp.dot(yagg, w_r[...], preferred_element_type=jnp.float32) + b_r[...]
        ae = jnp.dot(yagg, pe_r[...],
                     preferred_element_type=jnp.float32) + qe_r[...]
        y0_r[...] = jnp.concatenate([y[:, :128], ae], axis=-1)
        y1_r[...] = jnp.concatenate([y[:, 128:], ae], axis=-1)

    grid = (_NE // _BR,)
    return pl.pallas_call(
        body,
        grid=grid,
        in_specs=[pl.BlockSpec((2, _BR, 144), lambda i: (0, i, 0)),
                  pl.BlockSpec((128, 256), lambda i: (0, 0)),
                  pl.BlockSpec((1, 256), lambda i: (0, 0)),
                  pl.BlockSpec((128, 16), lambda i: (0, 0)),
                  pl.BlockSpec((1, 16), lambda i: (0, 0))],
        out_specs=[pl.BlockSpec((_BR, 144), lambda i: (i, 0)),
                   pl.BlockSpec((_BR, 144), lambda i: (i, 0))],
        out_shape=[jax.ShapeDtypeStruct((_NE, 144), jnp.float32),
                   jax.ShapeDtypeStruct((_NE, 144), jnp.float32)],
    )(aggP, Wcat, bcat, PE, qE)


def _tc_l2vert(numerP, W2, b2, oad):
    def body(n_r, w_r, b_r, ad_r, x2_r, av_r):
        blocks = []
        for p in range(4):
            full = n_r[2 * p] + n_r[2 * p + 1]
            half = p % 2
            for cch in range(2):
                lane = 128 + 2 * half + cch
                dcol = jnp.maximum(full[:, lane:lane + 1], 1e-12)
                blocks.append(_elu(full[:, 64 * cch: 64 * cch + 64] / dcol))
        out1 = jnp.concatenate(blocks, axis=-1)
        x2 = jnp.dot(out1, w_r[...], preferred_element_type=jnp.float32) + b_r[...]
        x2_r[...] = x2
        av = jnp.sum(x2 * ad_r[...], axis=-1, keepdims=True)
        av_r[...] = jnp.concatenate(
            [av, jnp.zeros((av.shape[0], 15), jnp.float32)], axis=-1)

    grid = (_NV // _BR,)
    return pl.pallas_call(
        body,
        grid=grid,
        in_specs=[pl.BlockSpec((8, _BR, 144), lambda i: (0, i, 0)),
                  pl.BlockSpec((512, 16), lambda i: (0, 0)),
                  pl.BlockSpec((1, 16), lambda i: (0, 0)),
                  pl.BlockSpec((1, 16), lambda i: (0, 0))],
        out_specs=[pl.BlockSpec((_BR, 16), lambda i: (i, 0)),
                   pl.BlockSpec((_BR, 16), lambda i: (i, 0))],
        out_shape=[jax.ShapeDtypeStruct((_NV, 16), jnp.float32),
                   jax.ShapeDtypeStruct((_NV, 16), jnp.float32)],
    )(numerP, W2, b2, oad)


def _tc_l2edge(agg2P, cntP, oae):
    def body(a_r, c_r, ae_w, yt_r):
        cnt = jnp.maximum(c_r[0, :, 0:1] + c_r[1, :, 0:1], 1.0)
        y2 = (a_r[0] + a_r[1]) / cnt
        ae = jnp.sum(y2 * ae_w[...], axis=-1, keepdims=True)
        yt_r[...] = jnp.concatenate(
            [y2, ae, jnp.zeros((ae.shape[0], 15), jnp.float32)], axis=-1)

    grid = (_NE // _BR,)
    return pl.pallas_call(
        body,
        grid=grid,
        in_specs=[pl.BlockSpec((2, _BR, 16), lambda i: (0, i, 0)),
                  pl.BlockSpec((2, _BR, 16), lambda i: (0, i, 0)),
                  pl.BlockSpec((1, 16), lambda i: (0, 0))],
        out_specs=pl.BlockSpec((_BR, 32), lambda i: (i, 0)),
        out_shape=jax.ShapeDtypeStruct((_NE, 32), jnp.float32),
    )(agg2P, cntP, oae)


def _tc_final(numer2P):
    def body(n_r, o_r):
        full = n_r[0] + n_r[1]
        den = jnp.maximum(full[:, 16:17], 1e-12)
        o_r[...] = _elu(full[:, :16] / den)

    grid = (_NV // _BR,)
    return pl.pallas_call(
        body,
        grid=grid,
        in_specs=[pl.BlockSpec((2, _BR, 32), lambda i: (0, i, 0))],
        out_specs=pl.BlockSpec((_BR, 16), lambda i: (i, 0)),
        out_shape=jax.ShapeDtypeStruct((_NV, _NCLS), jnp.float32),
    )(numer2P)


def kernel(x_list, hg, heads_theta_w, heads_theta_b, heads_att_e,
           heads_att_dst, out_theta_w, out_theta_b, out_att_e, out_att_dst):
    v_idx, e_idx = hg[0], hg[1]

    Wcat = jnp.concatenate([heads_theta_w[h] for h in range(_NH)], axis=1)
    bcat = heads_theta_b.reshape(1, _NH * _DHID)
    blkmask = jnp.kron(jnp.eye(_NH, dtype=jnp.float32),
                       jnp.ones((_DHID, 1), jnp.float32))
    BDd = jnp.pad(heads_att_dst.reshape(-1, 1) * blkmask, ((0, 0), (0, 12)))
    BDe = jnp.pad(heads_att_e.reshape(-1, 1) * blkmask, ((0, 0), (0, 12)))
    PV, qV, PE, qE = _tc_prep(Wcat, bcat, BDd, BDe)

    alphaV = _tc_matvec(x_list, PV, qV)

    ones_pad = jnp.concatenate(
        [jnp.ones((2, _NV, 1), jnp.float32),
         jnp.zeros((2, _NV, 15), jnp.float32)], axis=-1)
    x_aug = jnp.concatenate([x_list, ones_pad], axis=-1)

    v2e = _make_sc_multi(144, ((0, None, 1, 0),))
    numer_parts = []
    aggP0 = None
    for k in range(2):
        aggP = v2e((x_aug[k],), v_idx, e_idx, ())
        if aggP0 is None:
            aggP0 = aggP
        y0, y1 = _tc_edge(aggP, Wcat, bcat, PE, qE)
        for half, ytab in enumerate((y0, y1)):
            e2v = _make_sc_multi(144, ((0, 0, 2, 2 * half),))
            numer_parts.append(e2v((ytab,), e_idx, v_idx,
                                   (alphaV[k],))[:, :_NV])

    numerP = jnp.concatenate(numer_parts, axis=0)

    X2, aV2 = _tc_l2vert(numerP, out_theta_w, out_theta_b.reshape(1, -1),
                         out_att_dst.reshape(1, -1))
    v2e2 = _make_sc_multi(16, ((0, None, 1, 0),))
    agg2P = v2e2((X2,), v_idx, e_idx, ())
    ytab2 = _tc_l2edge(agg2P[:, :_NE], aggP0[:, :_NE, 128:144],
                       out_att_e.reshape(1, -1))
    e2v2 = _make_sc_multi(32, ((0, 0, 1, 0),))
    n2P = e2v2((ytab2,), e_idx, v_idx, (aV2,))
    return _tc_final(n2P[:, :_NV])
